# baseline (speedup 1.0000x reference)
import functools
import math

import jax
import jax.numpy as jnp
from jax import lax
from jax.experimental import pallas as pl
from jax.experimental.pallas import tpu as pltpu
from jax.experimental.pallas import tpu_sc as plsc

N = 10000
E = 160000
D = 256
H = 8
DH = 32
AUG = 288

NC = 2
NS = 16
NW = NC * NS
EP = 163840
EPW = EP // NW
EPS = EP // NS
C1 = 128
C2 = 128
G1 = C1 // 16
ROWS_PER_SUB = N // NS

RB = 400
GRID = N // RB


def _tc1_body(x_ref, cc_ref, wq_ref, wk_ref, wv_ref, bq_ref, bk_ref, bv_ref,
              lng_ref, lnb_ref, z_ref, s0_ref, s1_ref, s2_ref, t_ref,
              qaug_ref, kaug_ref, v0_ref, v1_ref):
    x = x_ref[...]
    m = jnp.mean(x, axis=-1, keepdims=True)
    v = jnp.mean((x - m) ** 2, axis=-1, keepdims=True)
    xn = (x - m) * lax.rsqrt(v + 1e-5) * lng_ref[...] + lnb_ref[...]
    delta = 0.5 * xn * (1.0 + lax.erf(xn / math.sqrt(2.0)))
    scale = 1.0 / math.sqrt(DH)
    q = (jnp.dot(delta, wq_ref[...], preferred_element_type=jnp.float32)
         + bq_ref[...]) * scale
    k = (jnp.dot(delta, wk_ref[...], preferred_element_type=jnp.float32)
         + bk_ref[...])
    vv = (jnp.dot(delta, wv_ref[...], preferred_element_type=jnp.float32)
          + bv_ref[...])
    a = jnp.dot(q, z_ref[...], preferred_element_type=jnp.float32)
    a0, a1, a2 = a[:, :8], a[:, 8:16], a[:, 16:24]
    cx = cc_ref[...][:, 0:1]
    cy = cc_ref[...][:, 1:2]
    b = a2 - a0 * cx - a1 * cy
    extq = (jnp.dot(a0, s0_ref[...], preferred_element_type=jnp.float32)
            + jnp.dot(a1, s1_ref[...], preferred_element_type=jnp.float32)
            + jnp.dot(b, s2_ref[...], preferred_element_type=jnp.float32))
    t0 = t_ref[...][0:1, :]
    t1 = t_ref[...][1:2, :]
    t2 = t_ref[...][2:3, :]
    extk = cx * t0 + cy * t1 + t2
    qaug_ref[:, :256] = q
    qaug_ref[:, 256:] = extq
    kaug_ref[:, :256] = k
    kaug_ref[:, 256:] = extk
    v0_ref[...] = vv[:, :128]
    v1_ref[...] = vv[:, 128:]


def _tc1(x, cc, wqt, wkt, wvt, bq, bk, bv, lng, lnb, z, s0, s1, s2, t):
    full = lambda shp: pl.BlockSpec(shp, lambda i: (0,) * len(shp))
    return pl.pallas_call(
        _tc1_body,
        grid=(GRID,),
        in_specs=[
            pl.BlockSpec((RB, 256), lambda i: (i, 0)),
            pl.BlockSpec((RB, 2), lambda i: (i, 0)),
            full((256, 256)), full((256, 256)), full((256, 256)),
            full((1, 256)), full((1, 256)), full((1, 256)),
            full((1, 256)), full((1, 256)),
            full((256, 24)),
            full((8, 32)), full((8, 32)), full((8, 32)),
            full((3, 32)),
        ],
        out_specs=[
            pl.BlockSpec((RB, AUG), lambda i: (i, 0)),
            pl.BlockSpec((RB, AUG), lambda i: (i, 0)),
            pl.BlockSpec((RB, 128), lambda i: (i, 0)),
            pl.BlockSpec((RB, 128), lambda i: (i, 0)),
        ],
        out_shape=[
            jax.ShapeDtypeStruct((N, AUG), jnp.float32),
            jax.ShapeDtypeStruct((N, AUG), jnp.float32),
            jax.ShapeDtypeStruct((N, 128), jnp.float32),
            jax.ShapeDtypeStruct((N, 128), jnp.float32),
        ],
    )(x, cc, wqt, wkt, wvt, bq, bk, bv, lng, lnb, z, s0, s1, s2, t)


_MESH = plsc.VectorSubcoreMesh(core_axis_name="c", subcore_axis_name="s",
                               num_cores=NC, num_subcores=NS)
_SC_PARAMS = pltpu.CompilerParams(use_tc_tiling_on_sc=False,
                                  needs_layout_passes=False)


@functools.partial(
    pl.kernel,
    out_type=jax.ShapeDtypeStruct((H, EP), jnp.float32),
    mesh=_MESH,
    scratch_types=[
        pltpu.VMEM((C1,), jnp.int32),
        pltpu.VMEM((C1,), jnp.int32),
        pltpu.VMEM((C1,), jnp.float32),
        pltpu.VMEM((C1, AUG), jnp.float32),
        pltpu.VMEM((C1, AUG), jnp.float32),
        pltpu.VMEM((H, C1), jnp.float32),
        pltpu.SemaphoreType.DMA,
    ],
    compiler_params=_SC_PARAMS,
)
def _sc_phase1(qaug_hbm, kaug_hbm, src_hbm, tgt_hbm, w_hbm, attn_hbm,
               sidx, tidx, wbuf, qrows, krows, attnb, sem):
    wid = lax.axis_index("s") * NC + lax.axis_index("c")
    base = wid * EPW

    def chunk_body(ci, carry):
        e0 = base + ci * C1
        pltpu.sync_copy(src_hbm.at[pl.ds(e0, C1)], sidx)
        pltpu.sync_copy(tgt_hbm.at[pl.ds(e0, C1)], tidx)
        pltpu.sync_copy(w_hbm.at[pl.ds(e0, C1)], wbuf)
        cq = pltpu.async_copy(qaug_hbm.at[sidx], qrows, sem)
        ck = pltpu.async_copy(kaug_hbm.at[tidx], krows, sem)
        cq.wait()
        ck.wait()

        def group_body(gi, carry2):
            i0 = gi * 16
            rows = i0 + lax.iota(jnp.int32, 16)
            for h in range(H):
                acc = jnp.zeros((16,), jnp.float32)
                for j in range(35):
                    d = DH * h + j if j < DH else 256 + 4 * h + (j - DH)
                    col = jnp.full((16,), d, jnp.int32)
                    qv = plsc.load_gather(qrows, [rows, col])
                    kv = plsc.load_gather(krows, [rows, col])
                    acc = acc + qv * kv
                wv = wbuf[pl.ds(i0, 16)]
                attnb[h, pl.ds(i0, 16)] = wv / (1.0 + jnp.exp(-acc))
            return carry2

        lax.fori_loop(0, G1, group_body, 0)
        pltpu.sync_copy(attnb, attn_hbm.at[:, pl.ds(e0, C1)])
        return carry

    lax.fori_loop(0, EPW // C1, chunk_body, 0)


@functools.partial(
    pl.kernel,
    out_type=jax.ShapeDtypeStruct((2, N, 128), jnp.float32),
    mesh=_MESH,
    scratch_types=[
        pltpu.VMEM((C2,), jnp.int32),
        pltpu.VMEM((C2,), jnp.int32),
        pltpu.VMEM((4, C2), jnp.float32),
        pltpu.VMEM((C2, 128), jnp.float32),
        pltpu.VMEM((C2, 128), jnp.float32),
        pltpu.VMEM_SHARED((N, 128), jnp.float32),
        pltpu.SemaphoreType.DMA,
    ],
    compiler_params=_SC_PARAMS,
)
def _sc_phase2(v_hbm, attn_hbm, src_hbm, tgt_hbm, out_hbm,
               sidx, tidx, attnb, vrows, msgs, agg, sem):
    c = lax.axis_index("c")
    s = lax.axis_index("s")

    def zrow(r, carry):
        for h8 in range(8):
            msgs[r, pl.ds(16 * h8, 16)] = jnp.zeros((16,), jnp.float32)
        return carry
    lax.fori_loop(0, C2, zrow, 0)
    r0 = s * ROWS_PER_SUB
    for j in range(ROWS_PER_SUB // 125):
        pltpu.sync_copy(msgs.at[pl.ds(0, 125)],
                        agg.at[pl.ds(r0 + j * 125, 125)])
    plsc.subcore_barrier()

    def chunk_body(ci, carry):
        e0 = s * EPS + ci * C2
        pltpu.sync_copy(src_hbm.at[pl.ds(e0, C2)], sidx)
        pltpu.sync_copy(tgt_hbm.at[pl.ds(e0, C2)], tidx)
        pltpu.sync_copy(attn_hbm.at[pl.ds(4 * c, 4), pl.ds(e0, C2)], attnb)
        pltpu.async_copy(v_hbm.at[c].at[tidx], vrows, sem).wait()

        def group_body(gi, carry2):
            i0 = gi * 16
            rows = i0 + lax.iota(jnp.int32, 16)
            for hh in range(4):
                a = attnb[hh, pl.ds(i0, 16)]
                for j in range(DH):
                    d = DH * hh + j
                    col = jnp.full((16,), d, jnp.int32)
                    vcol = plsc.load_gather(vrows, [rows, col])
                    plsc.store_scatter(msgs, [rows, col], a * vcol)
            return carry2

        lax.fori_loop(0, C2 // 16, group_body, 0)
        pltpu.sync_copy(msgs, agg.at[sidx], add=True)
        return carry

    lax.fori_loop(0, EPS // C2, chunk_body, 0)
    plsc.subcore_barrier()
    pltpu.sync_copy(agg.at[pl.ds(r0, ROWS_PER_SUB)],
                    out_hbm.at[c].at[pl.ds(r0, ROWS_PER_SUB)])


def _tc2_body(x_ref, a0_ref, a1_ref, w0_ref, w1_ref, o_ref):
    o_ref[...] = (x_ref[...]
                  + jnp.dot(a0_ref[0], w0_ref[...],
                            preferred_element_type=jnp.float32)
                  + jnp.dot(a1_ref[0], w1_ref[...],
                            preferred_element_type=jnp.float32))


def _tc2(x, agg, wot0, wot1):
    full = lambda shp: pl.BlockSpec(shp, lambda i: (0,) * len(shp))
    return pl.pallas_call(
        _tc2_body,
        grid=(GRID,),
        in_specs=[
            pl.BlockSpec((RB, 256), lambda i: (i, 0)),
            pl.BlockSpec((1, RB, 128), lambda i: (0, i, 0)),
            pl.BlockSpec((1, RB, 128), lambda i: (1, i, 0)),
            full((128, 256)), full((128, 256)),
        ],
        out_specs=pl.BlockSpec((RB, 256), lambda i: (i, 0)),
        out_shape=jax.ShapeDtypeStruct((N, 256), jnp.float32),
    )(x, agg, agg, wot0, wot1)


def kernel(in_feats, edge_ids, edge_weights, node_cxcy, ln_g, ln_b,
           Wq, bq, Wk, bk, Wv, bv, Wp, bp, Wo):
    f32 = jnp.float32
    didx = jnp.arange(D)
    hid = didx // DH
    Z0 = jnp.zeros((D, H), f32).at[didx, hid].set(Wp[:, 0])
    Z1 = jnp.zeros((D, H), f32).at[didx, hid].set(Wp[:, 1])
    Z2 = jnp.zeros((D, H), f32).at[didx, hid].set(bp)
    Z = jnp.concatenate([Z0, Z1, Z2], axis=1)
    harange = jnp.arange(H)
    S0 = jnp.zeros((H, 32), f32).at[harange, 4 * harange].set(1.0)
    S1 = jnp.zeros((H, 32), f32).at[harange, 4 * harange + 1].set(1.0)
    S2 = jnp.zeros((H, 32), f32).at[harange, 4 * harange + 2].set(1.0)
    t0 = jnp.zeros((32,), f32).at[4 * harange].set(1.0)
    t1 = jnp.zeros((32,), f32).at[4 * harange + 1].set(1.0)
    t2 = jnp.zeros((32,), f32).at[4 * harange + 2].set(1.0)
    T = jnp.stack([t0, t1, t2], axis=0)

    qaug, kaug, v0, v1 = _tc1(
        in_feats, node_cxcy, Wq.T, Wk.T, Wv.T,
        bq.reshape(1, D), bk.reshape(1, D), bv.reshape(1, D),
        ln_g.reshape(1, D), ln_b.reshape(1, D), Z, S0, S1, S2, T)

    pad = EP - E
    src = jnp.pad(edge_ids[0], (0, pad))
    tgt = jnp.pad(edge_ids[1], (0, pad))
    ew = jnp.pad(edge_weights, (0, pad))

    attn = _sc_phase1(qaug, kaug, src, tgt, ew)

    vsplit = jnp.stack([v0, v1], axis=0)
    agg = _sc_phase2(vsplit, attn, src, tgt)

    wot = Wo.T
    return _tc2(in_feats, agg, wot[:128], wot[128:])


# speedup vs baseline: 1.2633x; 1.2633x over previous
import functools
import math

import jax
import jax.numpy as jnp
from jax import lax
from jax.experimental import pallas as pl
from jax.experimental.pallas import tpu as pltpu
from jax.experimental.pallas import tpu_sc as plsc

N = 10000
E = 160000
D = 256
H = 8
DH = 32
AUG = 288

NC = 2
NS = 16
NW = NC * NS
EP = 163840
EPW = EP // NW
EPS = EP // NS
C1 = 64
NCH1 = EPW // C1
C2 = 64
NCH2 = EPS // C2
G1 = C1 // 16
ROWS_PER_SUB = N // NS

RB = 400
GRID = N // RB


def _tc1_body(x_ref, cc_ref, wq_ref, wk_ref, wv_ref, bq_ref, bk_ref, bv_ref,
              lng_ref, lnb_ref, z_ref, s0_ref, s1_ref, s2_ref, t_ref,
              qaug_ref, kaug_ref, v0_ref, v1_ref):
    x = x_ref[...]
    m = jnp.mean(x, axis=-1, keepdims=True)
    v = jnp.mean((x - m) ** 2, axis=-1, keepdims=True)
    xn = (x - m) * lax.rsqrt(v + 1e-5) * lng_ref[...] + lnb_ref[...]
    delta = 0.5 * xn * (1.0 + lax.erf(xn / math.sqrt(2.0)))
    scale = 1.0 / math.sqrt(DH)
    q = (jnp.dot(delta, wq_ref[...], preferred_element_type=jnp.float32)
         + bq_ref[...]) * scale
    k = (jnp.dot(delta, wk_ref[...], preferred_element_type=jnp.float32)
         + bk_ref[...])
    vv = (jnp.dot(delta, wv_ref[...], preferred_element_type=jnp.float32)
          + bv_ref[...])
    a = jnp.dot(q, z_ref[...], preferred_element_type=jnp.float32)
    a0, a1, a2 = a[:, :8], a[:, 8:16], a[:, 16:24]
    cx = cc_ref[...][:, 0:1]
    cy = cc_ref[...][:, 1:2]
    b = a2 - a0 * cx - a1 * cy
    extq = (jnp.dot(a0, s0_ref[...], preferred_element_type=jnp.float32)
            + jnp.dot(a1, s1_ref[...], preferred_element_type=jnp.float32)
            + jnp.dot(b, s2_ref[...], preferred_element_type=jnp.float32))
    t0 = t_ref[...][0:1, :]
    t1 = t_ref[...][1:2, :]
    t2 = t_ref[...][2:3, :]
    extk = cx * t0 + cy * t1 + t2
    qaug_ref[:, :256] = q
    qaug_ref[:, 256:] = extq
    kaug_ref[:, :256] = k
    kaug_ref[:, 256:] = extk
    v0_ref[...] = vv[:, :128]
    v1_ref[...] = vv[:, 128:]


def _tc1(x, cc, wqt, wkt, wvt, bq, bk, bv, lng, lnb, z, s0, s1, s2, t):
    full = lambda shp: pl.BlockSpec(shp, lambda i: (0,) * len(shp))
    return pl.pallas_call(
        _tc1_body,
        grid=(GRID,),
        in_specs=[
            pl.BlockSpec((RB, 256), lambda i: (i, 0)),
            pl.BlockSpec((RB, 2), lambda i: (i, 0)),
            full((256, 256)), full((256, 256)), full((256, 256)),
            full((1, 256)), full((1, 256)), full((1, 256)),
            full((1, 256)), full((1, 256)),
            full((256, 24)),
            full((8, 32)), full((8, 32)), full((8, 32)),
            full((3, 32)),
        ],
        out_specs=[
            pl.BlockSpec((RB, AUG), lambda i: (i, 0)),
            pl.BlockSpec((RB, AUG), lambda i: (i, 0)),
            pl.BlockSpec((RB, 128), lambda i: (i, 0)),
            pl.BlockSpec((RB, 128), lambda i: (i, 0)),
        ],
        out_shape=[
            jax.ShapeDtypeStruct((N, AUG), jnp.float32),
            jax.ShapeDtypeStruct((N, AUG), jnp.float32),
            jax.ShapeDtypeStruct((N, 128), jnp.float32),
            jax.ShapeDtypeStruct((N, 128), jnp.float32),
        ],
    )(x, cc, wqt, wkt, wvt, bq, bk, bv, lng, lnb, z, s0, s1, s2, t)


_MESH = plsc.VectorSubcoreMesh(core_axis_name="c", subcore_axis_name="s",
                               num_cores=NC, num_subcores=NS)
_SC_PARAMS = pltpu.CompilerParams(use_tc_tiling_on_sc=False,
                                  needs_layout_passes=False)


@functools.partial(
    pl.kernel,
    out_type=jax.ShapeDtypeStruct((H, EP), jnp.float32),
    mesh=_MESH,
    scratch_types=[
        pltpu.VMEM((NCH1, C1), jnp.int32),
        pltpu.VMEM((NCH1, C1), jnp.int32),
        pltpu.VMEM((EPW,), jnp.float32),
        pltpu.VMEM((C1, AUG), jnp.float32),
        pltpu.VMEM((C1, AUG), jnp.float32),
        pltpu.VMEM((C1, AUG), jnp.float32),
        pltpu.VMEM((C1, AUG), jnp.float32),
        pltpu.VMEM((H, C1), jnp.float32),
        pltpu.VMEM((H, C1), jnp.float32),
        pltpu.SemaphoreType.DMA,
        pltpu.SemaphoreType.DMA,
        pltpu.SemaphoreType.DMA,
    ],
    compiler_params=_SC_PARAMS,
)
def _sc_phase1(qaug_hbm, kaug_hbm, src_hbm, tgt_hbm, w_hbm, attn_hbm,
               sidx, tidx, wbuf, qr0, qr1, kr0, kr1, at0, at1,
               sg0, sg1, sout):
    wid = lax.axis_index("s") * NC + lax.axis_index("c")
    base = wid * EPW

    pltpu.sync_copy(src_hbm.at[wid], sidx)
    pltpu.sync_copy(tgt_hbm.at[wid], tidx)
    pltpu.sync_copy(w_hbm.at[wid], wbuf)

    def fire(ci, qb, kb, sb):
        pltpu.async_copy(qaug_hbm.at[sidx.at[ci]], qb, sb)
        pltpu.async_copy(kaug_hbm.at[tidx.at[ci]], kb, sb)

    def drain(ci, qb, kb, sb):
        pltpu.make_async_copy(qaug_hbm.at[sidx.at[ci]], qb, sb).wait()
        pltpu.make_async_copy(kaug_hbm.at[tidx.at[ci]], kb, sb).wait()

    def compute(ci, qb, kb, atb, first_out):
        def group_body(gi, carry2):
            i0 = gi * 16
            rows = i0 + lax.iota(jnp.int32, 16)
            for h in range(H):
                acc = jnp.zeros((16,), jnp.float32)
                for j in range(35):
                    d = DH * h + j if j < DH else 256 + 4 * h + (j - DH)
                    col = jnp.full((16,), d, jnp.int32)
                    qv = plsc.load_gather(qb, [rows, col])
                    kv = plsc.load_gather(kb, [rows, col])
                    acc = acc + qv * kv
                wv = wbuf[pl.ds(ci * C1 + i0, 16)]
                atb[h, pl.ds(i0, 16)] = wv / (1.0 + jnp.exp(-acc))
            return carry2

        lax.fori_loop(0, G1, group_body, 0)
        @pl.when(jnp.logical_not(first_out))
        def _():
            pltpu.make_async_copy(atb, attn_hbm.at[:, pl.ds(0, C1)],
                                  sout).wait()
        pltpu.async_copy(atb, attn_hbm.at[:, pl.ds(base + ci * C1, C1)], sout)

    fire(0, qr0, kr0, sg0)

    def outer(co, carry):
        ci0 = 2 * co
        fire(ci0 + 1, qr1, kr1, sg1)
        drain(ci0, qr0, kr0, sg0)
        compute(ci0, qr0, kr0, at0, co == 0)

        @pl.when(ci0 + 2 < NCH1)
        def _():
            fire(ci0 + 2, qr0, kr0, sg0)
        drain(ci0 + 1, qr1, kr1, sg1)
        compute(ci0 + 1, qr1, kr1, at1, co == 0)
        return carry

    lax.fori_loop(0, NCH1 // 2, outer, 0)
    pltpu.make_async_copy(at0, attn_hbm.at[:, pl.ds(0, C1)], sout).wait()
    pltpu.make_async_copy(at1, attn_hbm.at[:, pl.ds(0, C1)], sout).wait()


@functools.partial(
    pl.kernel,
    out_type=jax.ShapeDtypeStruct((2, N, 128), jnp.float32),
    mesh=_MESH,
    scratch_types=[
        pltpu.VMEM((NCH2, C2), jnp.int32),
        pltpu.VMEM((C2,), jnp.int32),
        pltpu.VMEM((C2,), jnp.int32),
        pltpu.VMEM((4, C2), jnp.float32),
        pltpu.VMEM((4, C2), jnp.float32),
        pltpu.VMEM((C2, 128), jnp.float32),
        pltpu.VMEM((C2, 128), jnp.float32),
        pltpu.VMEM((C2, 128), jnp.float32),
        pltpu.VMEM((C2, 128), jnp.float32),
        pltpu.VMEM_SHARED((N, 128), jnp.float32),
        pltpu.SemaphoreType.DMA,
        pltpu.SemaphoreType.DMA,
        pltpu.SemaphoreType.DMA,
        pltpu.SemaphoreType.DMA,
    ],
    compiler_params=_SC_PARAMS,
)
def _sc_phase2(v_hbm, attn_hbm, src_hbm, tgt_hbm, out_hbm,
               sidx, tid0, tid1, at0, at1, vr0, vr1, ms0, ms1, agg,
               sg0, sg1, stid, ssc):
    c = lax.axis_index("c")
    s = lax.axis_index("s")
    base = s * EPS

    def zrow(r, carry):
        for h8 in range(8):
            ms0[r, pl.ds(16 * h8, 16)] = jnp.zeros((16,), jnp.float32)
        return carry
    lax.fori_loop(0, C2, zrow, 0)
    r0 = s * ROWS_PER_SUB
    for j in range(ROWS_PER_SUB // C2):
        pltpu.sync_copy(ms0.at[pl.ds(0, C2)],
                        agg.at[pl.ds(r0 + j * C2, C2)])
    rem = ROWS_PER_SUB - (ROWS_PER_SUB // C2) * C2
    pltpu.sync_copy(ms0.at[pl.ds(0, rem)],
                    agg.at[pl.ds(r0 + (ROWS_PER_SUB // C2) * C2, rem)])
    plsc.subcore_barrier()

    pltpu.sync_copy(src_hbm.at[s], sidx)

    def fire_tid(ci, tb):
        pltpu.async_copy(tgt_hbm.at[s, pl.ds(ci * C2, C2)], tb, stid)

    def drain_tid(ci, tb):
        pltpu.make_async_copy(tgt_hbm.at[s, pl.ds(ci * C2, C2)],
                              tb, stid).wait()

    def fire(ci, tb, vb, ab, sb):
        pltpu.async_copy(v_hbm.at[c].at[tb], vb, sb)
        pltpu.async_copy(
            attn_hbm.at[pl.ds(4 * c, 4), pl.ds(base + ci * C2, C2)], ab, sb)

    def drain(ci, tb, vb, ab, sb):
        pltpu.make_async_copy(v_hbm.at[c].at[tb], vb, sb).wait()
        pltpu.make_async_copy(
            attn_hbm.at[pl.ds(4 * c, 4), pl.ds(base + ci * C2, C2)],
            ab, sb).wait()

    def compute(ci, vb, ab, mb, first_sc):
        @pl.when(jnp.logical_not(first_sc))
        def _():
            pltpu.make_async_copy(mb, agg.at[sidx.at[ci]], ssc).wait()

        def group_body(gi, carry2):
            i0 = gi * 16
            rows = i0 + lax.iota(jnp.int32, 16)
            for hh in range(4):
                a = ab[hh, pl.ds(i0, 16)]
                for j in range(DH):
                    d = DH * hh + j
                    col = jnp.full((16,), d, jnp.int32)
                    vcol = plsc.load_gather(vb, [rows, col])
                    plsc.store_scatter(mb, [rows, col], a * vcol)
            return carry2

        lax.fori_loop(0, C2 // 16, group_body, 0)
        pltpu.async_copy(mb, agg.at[sidx.at[ci]], ssc, add=True)

    pltpu.sync_copy(tgt_hbm.at[s, pl.ds(0, C2)], tid0)
    fire(0, tid0, vr0, at0, sg0)
    pltpu.sync_copy(tgt_hbm.at[s, pl.ds(C2, C2)], tid1)
    fire(1, tid1, vr1, at1, sg1)

    def outer(co, carry):
        ci0 = 2 * co
        drain(ci0, tid0, vr0, at0, sg0)

        @pl.when(ci0 + 2 < NCH2)
        def _():
            fire_tid(ci0 + 2, tid0)
        compute(ci0, vr0, at0, ms0, co == 0)

        @pl.when(ci0 + 2 < NCH2)
        def _():
            drain_tid(ci0 + 2, tid0)
            fire(ci0 + 2, tid0, vr0, at0, sg0)

        drain(ci0 + 1, tid1, vr1, at1, sg1)

        @pl.when(ci0 + 3 < NCH2)
        def _():
            fire_tid(ci0 + 3, tid1)
        compute(ci0 + 1, vr1, at1, ms1, co == 0)

        @pl.when(ci0 + 3 < NCH2)
        def _():
            drain_tid(ci0 + 3, tid1)
            fire(ci0 + 3, tid1, vr1, at1, sg1)
        return carry

    lax.fori_loop(0, NCH2 // 2, outer, 0)
    pltpu.make_async_copy(ms0, agg.at[sidx.at[0]], ssc).wait()
    pltpu.make_async_copy(ms1, agg.at[sidx.at[0]], ssc).wait()
    plsc.subcore_barrier()
    pltpu.sync_copy(agg.at[pl.ds(r0, ROWS_PER_SUB)],
                    out_hbm.at[c].at[pl.ds(r0, ROWS_PER_SUB)])


def _tc2_body(x_ref, a0_ref, a1_ref, w0_ref, w1_ref, o_ref):
    o_ref[...] = (x_ref[...]
                  + jnp.dot(a0_ref[0], w0_ref[...],
                            preferred_element_type=jnp.float32)
                  + jnp.dot(a1_ref[0], w1_ref[...],
                            preferred_element_type=jnp.float32))


def _tc2(x, agg, wot0, wot1):
    full = lambda shp: pl.BlockSpec(shp, lambda i: (0,) * len(shp))
    return pl.pallas_call(
        _tc2_body,
        grid=(GRID,),
        in_specs=[
            pl.BlockSpec((RB, 256), lambda i: (i, 0)),
            pl.BlockSpec((1, RB, 128), lambda i: (0, i, 0)),
            pl.BlockSpec((1, RB, 128), lambda i: (1, i, 0)),
            full((128, 256)), full((128, 256)),
        ],
        out_specs=pl.BlockSpec((RB, 256), lambda i: (i, 0)),
        out_shape=jax.ShapeDtypeStruct((N, 256), jnp.float32),
    )(x, agg, agg, wot0, wot1)


def kernel(in_feats, edge_ids, edge_weights, node_cxcy, ln_g, ln_b,
           Wq, bq, Wk, bk, Wv, bv, Wp, bp, Wo):
    f32 = jnp.float32
    didx = jnp.arange(D)
    hid = didx // DH
    Z0 = jnp.zeros((D, H), f32).at[didx, hid].set(Wp[:, 0])
    Z1 = jnp.zeros((D, H), f32).at[didx, hid].set(Wp[:, 1])
    Z2 = jnp.zeros((D, H), f32).at[didx, hid].set(bp)
    Z = jnp.concatenate([Z0, Z1, Z2], axis=1)
    harange = jnp.arange(H)
    S0 = jnp.zeros((H, 32), f32).at[harange, 4 * harange].set(1.0)
    S1 = jnp.zeros((H, 32), f32).at[harange, 4 * harange + 1].set(1.0)
    S2 = jnp.zeros((H, 32), f32).at[harange, 4 * harange + 2].set(1.0)
    t0 = jnp.zeros((32,), f32).at[4 * harange].set(1.0)
    t1 = jnp.zeros((32,), f32).at[4 * harange + 1].set(1.0)
    t2 = jnp.zeros((32,), f32).at[4 * harange + 2].set(1.0)
    T = jnp.stack([t0, t1, t2], axis=0)

    qaug, kaug, v0, v1 = _tc1(
        in_feats, node_cxcy, Wq.T, Wk.T, Wv.T,
        bq.reshape(1, D), bk.reshape(1, D), bv.reshape(1, D),
        ln_g.reshape(1, D), ln_b.reshape(1, D), Z, S0, S1, S2, T)

    pad = EP - E
    src = jnp.pad(edge_ids[0], (0, pad))
    tgt = jnp.pad(edge_ids[1], (0, pad))
    ew = jnp.pad(edge_weights, (0, pad))

    attn = _sc_phase1(qaug, kaug,
                      src.reshape(NW, NCH1, C1), tgt.reshape(NW, NCH1, C1),
                      ew.reshape(NW, EPW))

    vsplit = jnp.stack([v0, v1], axis=0)
    agg = _sc_phase2(vsplit, attn,
                     src.reshape(NS, NCH2, C2), tgt.reshape(NS, EPS))

    wot = Wo.T
    return _tc2(in_feats, agg, wot[:128], wot[128:])
